# baseline (speedup 1.0000x reference)
import functools

import jax
import jax.numpy as jnp
from jax import lax
from jax.experimental import pallas as pl
from jax.experimental.pallas import tpu as pltpu
from jax.experimental.pallas import tpu_sc as plsc

B, L = 4096, 200
VOCAB, DIM = 1000000, 64
HALF = 100
NCHUNK = (B * L) // HALF
LANES = 16
VECS_PER_ROW = DIM // LANES


@functools.partial(jax.jit, static_argnums=())
def _sc_embed(seq2d, tgt_table, pos3d):
    info = plsc.get_sparse_core_info()
    nc, ns = info.num_cores, info.num_subcores
    nw = nc * ns
    chunks_per_w = NCHUNK // nw

    mesh = plsc.VectorSubcoreMesh(core_axis_name="c", subcore_axis_name="s")

    @functools.partial(
        pl.kernel,
        mesh=mesh,
        out_type=jax.ShapeDtypeStruct((NCHUNK, HALF, DIM), jnp.float32),
        scratch_types=[
            pltpu.VMEM((HALF,), jnp.int32),
            pltpu.VMEM((HALF, DIM), jnp.float32),
            pltpu.VMEM((2, HALF, DIM), jnp.float32),
            pltpu.SemaphoreType.DMA,
        ],
        compiler_params=pltpu.CompilerParams(use_tc_tiling_on_sc=False),
    )
    def k(seq_hbm, table_hbm, pos_hbm, out_hbm, idx_v, rows_v, pos_v, sem):
        wid = lax.axis_index("s") * nc + lax.axis_index("c")
        c0 = wid * chunks_per_w
        pltpu.sync_copy(pos_hbm, pos_v)

        def body(i, carry):
            c = c0 + i
            h = lax.rem(i, 2)
            pltpu.sync_copy(seq_hbm.at[c], idx_v)
            pltpu.async_copy(table_hbm.at[idx_v], rows_v, sem).wait()

            def add_row(r, carry2):
                for kk in range(VECS_PER_ROW):
                    sl = pl.ds(kk * LANES, LANES)
                    rows_v[r, sl] = rows_v[r, sl] + pos_v[h, r, sl]
                return carry2

            lax.fori_loop(0, HALF, add_row, 0, unroll=2)
            pltpu.sync_copy(rows_v, out_hbm.at[c])
            return carry

        lax.fori_loop(0, chunks_per_w, body, 0)

    return k(seq2d, tgt_table, pos3d)


def kernel(seq, tgt_table, pos_table):
    seq2d = seq.reshape(NCHUNK, HALF)
    pos3d = pos_table[:L].reshape(2, HALF, DIM)
    out = _sc_embed(seq2d, tgt_table, pos3d)
    return out.reshape(B, L, DIM)


# speedup vs baseline: 1.2672x; 1.2672x over previous
import functools

import jax
import jax.numpy as jnp
from jax import lax
from jax.experimental import pallas as pl
from jax.experimental.pallas import tpu as pltpu
from jax.experimental.pallas import tpu_sc as plsc

B, L = 4096, 200
VOCAB, DIM = 1000000, 64
HALF = 100
NCHUNK = (B * L) // HALF
LANES = 16
VECS_PER_ROW = DIM // LANES
NIDX = 8
NROW = 4
PREF = 4


def _sc_embed(seq2d, tgt_table, pos3d):
    info = plsc.get_sparse_core_info()
    nc, ns = info.num_cores, info.num_subcores
    nw = nc * ns
    chunks_per_w = NCHUNK // nw

    mesh = plsc.VectorSubcoreMesh(core_axis_name="c", subcore_axis_name="s")

    scratch = (
        [pltpu.VMEM((HALF,), jnp.int32)] * NIDX
        + [pltpu.VMEM((HALF, DIM), jnp.float32)] * NROW
        + [pltpu.VMEM((2, HALF, DIM), jnp.float32)]
        + [pltpu.SemaphoreType.DMA] * (NIDX + NROW + NROW)
    )

    @functools.partial(
        pl.kernel,
        mesh=mesh,
        out_type=jax.ShapeDtypeStruct((NCHUNK, HALF, DIM), jnp.float32),
        scratch_types=scratch,
        compiler_params=pltpu.CompilerParams(use_tc_tiling_on_sc=False),
    )
    def k(seq_hbm, table_hbm, pos_hbm, out_hbm, *refs):
        idx_bufs = refs[0:NIDX]
        rows_bufs = refs[NIDX:NIDX + NROW]
        pos_v = refs[NIDX + NROW]
        sems = refs[NIDX + NROW + 1:]
        sem_i = sems[0:NIDX]
        sem_g = sems[NIDX:NIDX + NROW]
        sem_s = sems[NIDX + NROW:]

        wid = lax.axis_index("s") * nc + lax.axis_index("c")
        c0 = wid * chunks_per_w
        last_c = NCHUNK - 1

        pltpu.sync_copy(pos_hbm, pos_v)

        def idx_start(slot, c):
            cc = lax.min(c, last_c)
            pltpu.async_copy(seq_hbm.at[cc], idx_bufs[slot], sem_i[slot])

        def idx_wait(slot):
            pltpu.make_async_copy(
                seq_hbm.at[c0], idx_bufs[slot], sem_i[slot]).wait()

        def gather_start(slot, b):
            pltpu.async_copy(
                table_hbm.at[idx_bufs[slot]], rows_bufs[b], sem_g[b])

        def gather_wait(slot, b):
            pltpu.make_async_copy(
                table_hbm.at[idx_bufs[slot]], rows_bufs[b], sem_g[b]).wait()

        def store_start(b, c):
            pltpu.async_copy(rows_bufs[b], out_hbm.at[c], sem_s[b])

        def store_wait(b):
            pltpu.make_async_copy(
                rows_bufs[b], out_hbm.at[c0], sem_s[b]).wait()

        def add_pos(b, h):
            buf = rows_bufs[b]

            def add_row(r, carry):
                for kk in range(VECS_PER_ROW):
                    sl = pl.ds(kk * LANES, LANES)
                    buf[r, sl] = buf[r, sl] + pos_v[h, r, sl]
                return carry

            lax.fori_loop(0, HALF, add_row, 0, unroll=4)

        for j in range(PREF):
            idx_start(j, c0 + j)
        for j in range(NIDX):
            idx_wait(j)
            if j >= NROW:
                store_wait(j % NROW)
            gather_start(j, j % NROW)
            idx_start((j + PREF) % NIDX, c0 + j + PREF)
            if j >= 1:
                bp = (j - 1) % NROW
                gather_wait((j - 1) % NIDX, bp)
                add_pos(bp, (j - 1) & 1)
                store_start(bp, c0 + j - 1)

        def group(g, carry):
            for j in range(NIDX):
                i = g * NIDX + j
                c = c0 + i
                b = j % NROW
                idx_wait(j)
                store_wait(b)
                gather_start(j, b)
                idx_start((j + PREF) % NIDX, c + PREF)
                bp = (j - 1) % NROW
                gather_wait((j - 1) % NIDX, bp)
                add_pos(bp, (j - 1) & 1)
                store_start(bp, c - 1)
            return carry

        lax.fori_loop(1, chunks_per_w // NIDX, group, 0)

        last_i = chunks_per_w - 1
        bl = last_i % NROW
        gather_wait(last_i % NIDX, bl)
        add_pos(bl, last_i & 1)
        store_start(bl, c0 + last_i)
        for t in range(PREF):
            idx_wait(t)
        for b in range(NROW):
            store_wait(b)

    return k(seq2d, tgt_table, pos3d)


def kernel(seq, tgt_table, pos_table):
    seq2d = seq.reshape(NCHUNK, HALF)
    pos3d = pos_table[:L].reshape(2, HALF, DIM)
    out = _sc_embed(seq2d, tgt_table, pos3d)
    return out.reshape(B, L, DIM)
